# Initial kernel scaffold; baseline (speedup 1.0000x reference)
import jax
import jax.numpy as jnp
from jax.experimental import pallas as pl


def kernel(input_seq, input_timestamp, tgt_idx, dyn_times, dyn_node_emb, time_emb, pos_emb, W_q, W_k, W_v, W_o, lin1_w, lin1_b, lin2_w, lin2_b, ln1_s, ln1_b, ln2_s, ln2_b, out_w, out_b):
    raise NotImplementedError("write your pallas kernel here")


# speedup vs baseline: 4.1284x; 4.1284x over previous
import functools

import jax
import jax.numpy as jnp
from jax import lax
from jax.experimental import pallas as pl
from jax.experimental.pallas import tpu as pltpu
from jax.experimental.pallas import tpu_sc as plsc

_B = 16
_L = 49
_T = 8
_D = 64
_POS = 8
_DIN = _D + _POS
_H = 8
_DK = 64
_USERS = 100000
_STEP = 5
_NEG = -(2.0 ** 32) + 1.0

_ROWS = _T * _B * _L
_NW = 32
_CHUNK = 104
_PER_W = 208
_ROWS_PAD = _NW * _PER_W


def _sc_gather(table2d, idx):
    mesh = plsc.VectorSubcoreMesh(core_axis_name="c", subcore_axis_name="s")

    @functools.partial(
        pl.kernel,
        mesh=mesh,
        out_type=jax.ShapeDtypeStruct((_ROWS_PAD, _D), jnp.float32),
        compiler_params=pltpu.CompilerParams(use_tc_tiling_on_sc=False),
        scratch_types=[
            pltpu.VMEM((_CHUNK,), jnp.int32),
            pltpu.VMEM((_CHUNK, _D), jnp.float32),
            pltpu.SemaphoreType.DMA,
        ],
    )
    def gk(table_hbm, idx_hbm, out_hbm, idx_v, rows_v, sem):
        wid = lax.axis_index("s") * 2 + lax.axis_index("c")
        base = wid * _PER_W
        for c in range(_PER_W // _CHUNK):
            off = base + c * _CHUNK
            pltpu.sync_copy(idx_hbm.at[pl.ds(off, _CHUNK)], idx_v)
            pltpu.async_copy(table_hbm.at[idx_v], rows_v, sem).wait()
            pltpu.sync_copy(rows_v, out_hbm.at[pl.ds(off, _CHUNK)])

    return gk(table2d, idx)


def _attn_body(ts_ref, times_ref, dyu_ref, temb_ref, posl_ref, seq_ref,
               wq_ref, wk_ref, wv_ref, wo_ref,
               l1w_ref, l1b_ref, l2w_ref, l2b_ref,
               s1_ref, b1_ref, s2_ref, b2_ref, out_ref):
    f32 = jnp.float32
    ts = ts_ref[...]
    times = times_ref[...]
    col = lax.broadcasted_iota(jnp.int32, (_B, _L), 1)
    valid = ts >= 0
    sentinel = jnp.int32(-2 ** 31)
    cur = jnp.max(times)
    oh_rows = []
    for c in range((_L + _STEP - 1) // _STEP):
        lo, hi = _STEP * c, min(_STEP * c + _STEP, _L)
        sel = (col >= lo) & (col < hi) & valid
        has_valid = jnp.any(sel)
        cmax = jnp.max(jnp.where(sel, ts, sentinel))
        cur = jnp.where(has_valid, cmax, cur)
        cnt = jnp.sum((times <= cur).astype(jnp.int32))
        val = jnp.where(cnt > 0, cnt - 1, jnp.int32(_T - 1))
        oh_rows.append(
            (lax.broadcasted_iota(jnp.int32, (1, _T), 1) == val).astype(f32))
    oh10 = jnp.concatenate(oh_rows, axis=0)
    exp_c = (lax.broadcasted_iota(jnp.int32, (_L, 10), 0) // _STEP
             == lax.broadcasted_iota(jnp.int32, (_L, 10), 1)).astype(f32)
    t_sel = jnp.dot(oh10, temb_ref[...], preferred_element_type=f32)
    t_l = jnp.dot(exp_c, t_sel, preferred_element_type=f32)

    dyu = dyu_ref[...].reshape(_T, _L, _D)
    affs = [jnp.sum(t_l * dyu[t], axis=-1, keepdims=True) * (1.0 / (_D ** 0.5))
            for t in range(_T)]
    aff = jnp.concatenate(affs, axis=1)
    aff = aff - jnp.max(aff, axis=1, keepdims=True)
    ea = jnp.exp(aff)
    alpha = ea / jnp.sum(ea, axis=1, keepdims=True)
    dyemb = alpha[:, 0:1] * dyu[0]
    for t in range(1, _T):
        dyemb = dyemb + alpha[:, t:t + 1] * dyu[t]

    x = jnp.concatenate([dyemb, posl_ref[...]], axis=-1)

    seqv = seq_ref[...].reshape(1, _L)
    padmask = seqv == 0
    qi = lax.broadcasted_iota(jnp.int32, (_L, _L), 0)
    ki = lax.broadcasted_iota(jnp.int32, (_L, _L), 1)
    fullmask = (ki > qi) | padmask
    scale = 1.0 / (_DK ** 0.5 + 1e-6)
    vatt = jnp.zeros((_L, _DIN), f32)
    for h in range(_H):
        qh = jnp.dot(x, wq_ref[h], preferred_element_type=f32)
        kh = jnp.dot(x, wk_ref[h], preferred_element_type=f32)
        vh = jnp.dot(x, wv_ref[h], preferred_element_type=f32)
        sc = lax.dot_general(qh, kh, (((1,), (1,)), ((), ())),
                             preferred_element_type=f32) * scale
        sc = jnp.where(fullmask, f32(_NEG), sc)
        sc = sc - jnp.max(sc, axis=-1, keepdims=True)
        es = jnp.exp(sc)
        attn = es / jnp.sum(es, axis=-1, keepdims=True)
        ctx = jnp.dot(attn, vh, preferred_element_type=f32)
        vatt = vatt + jnp.dot(ctx, wo_ref[h], preferred_element_type=f32)

    def ln(v, s, b):
        mu = jnp.mean(v, axis=-1, keepdims=True)
        var = jnp.mean((v - mu) ** 2, axis=-1, keepdims=True)
        return (v - mu) / jnp.sqrt(var + 1e-5) * s + b

    x1 = ln(x + vatt, s1_ref[...], b1_ref[...])
    ffn = lax.dot_general(x1, l1w_ref[...], (((1,), (1,)), ((), ())),
                          preferred_element_type=f32) + l1b_ref[...]
    ffn = jnp.maximum(ffn, 0.0)
    ffn = lax.dot_general(ffn, l2w_ref[...], (((1,), (1,)), ((), ())),
                          preferred_element_type=f32) + l2b_ref[...]
    out_ref[...] = ln(x1 + ffn, s2_ref[...], b2_ref[...]).reshape(1, _L, _DIN)


_VT = 2048
_NV = (_USERS + _VT - 1) // _VT


def _proj_body(att_ref, w_ref, b_ref, seq_ref, out_ref):
    f32 = jnp.float32
    vt = pl.program_id(0)
    att = att_ref[0]
    logits = lax.dot_general(att, w_ref[...], (((1,), (1,)), ((), ())),
                             preferred_element_type=f32) + b_ref[...]
    colv = lax.broadcasted_iota(jnp.int32, (_L, _VT), 1) + vt * _VT
    eq = (colv == seq_ref[0]).astype(f32)
    qi = lax.broadcasted_iota(jnp.int32, (_L, _L), 0)
    ki = lax.broadcasted_iota(jnp.int32, (_L, _L), 1)
    tril = (ki <= qi).astype(f32)
    pref = jnp.dot(tril, eq, preferred_element_type=f32)
    masked = (pref > 0.5) | (colv == 0)
    out_ref[...] = (logits + jnp.where(masked, f32(-1000.0), f32(0.0))
                    ).reshape(1, _L, _VT)


def kernel(input_seq, input_timestamp, tgt_idx, dyn_times, dyn_node_emb,
           time_emb, pos_emb, W_q, W_k, W_v, W_o, lin1_w, lin1_b, lin2_w,
           lin2_b, ln1_s, ln1_b, ln2_s, ln2_b, out_w, out_b):
    f32 = jnp.float32
    seq = input_seq[:, :-1].astype(jnp.int32)
    ts = input_timestamp[:, :-1].astype(jnp.int32)

    table2d = dyn_node_emb.reshape(_T * _USERS, _D)
    seq_flat = seq.reshape(-1)
    idx = (jnp.arange(_T, dtype=jnp.int32)[:, None] * _USERS
           + seq_flat[None, :]).reshape(-1)
    idx = jnp.concatenate(
        [idx, jnp.zeros((_ROWS_PAD - _ROWS,), jnp.int32)])
    rows = _sc_gather(table2d, idx)
    dyu4 = rows[:_ROWS].reshape(_T, _B, _L, _D)

    wq_h = W_q.reshape(_DIN, _H, _DK).transpose(1, 0, 2)
    wk_h = W_k.reshape(_DIN, _H, _DK).transpose(1, 0, 2)
    wv_h = W_v.reshape(_DIN, _H, _DK).transpose(1, 0, 2)
    wo_h = W_o.reshape(_H, _DK, _DIN)
    seq3 = seq.reshape(_B, 1, _L)
    full = lambda shape: pl.BlockSpec(shape, lambda b: (0,) * len(shape))
    att_out = pl.pallas_call(
        _attn_body,
        grid=(_B,),
        in_specs=[
            full((_B, _L)),
            full((1, _T)),
            pl.BlockSpec((_T, 1, _L, _D), lambda b: (0, b, 0, 0)),
            full((_T, _D)),
            full((_L, _POS)),
            pl.BlockSpec((1, 1, _L), lambda b: (b, 0, 0)),
            full((_H, _DIN, _DK)), full((_H, _DIN, _DK)),
            full((_H, _DIN, _DK)), full((_H, _DK, _DIN)),
            full((_DIN, _DIN)), full((1, _DIN)),
            full((_DIN, _DIN)), full((1, _DIN)),
            full((1, _DIN)), full((1, _DIN)),
            full((1, _DIN)), full((1, _DIN)),
        ],
        out_specs=pl.BlockSpec((1, _L, _DIN), lambda b: (b, 0, 0)),
        out_shape=jax.ShapeDtypeStruct((_B, _L, _DIN), f32),
    )(ts, dyn_times.reshape(1, _T).astype(jnp.int32),
      dyu4.reshape(_T, _B, _L, _D), time_emb, pos_emb[:_L], seq3,
      wq_h, wk_h, wv_h, wo_h,
      lin1_w, lin1_b.reshape(1, _DIN), lin2_w, lin2_b.reshape(1, _DIN),
      ln1_s.reshape(1, _DIN), ln1_b.reshape(1, _DIN),
      ln2_s.reshape(1, _DIN), ln2_b.reshape(1, _DIN))

    seq_t = seq.reshape(_B, _L, 1)
    out3 = pl.pallas_call(
        _proj_body,
        grid=(_NV, _B),
        in_specs=[
            pl.BlockSpec((1, _L, _DIN), lambda v, b: (b, 0, 0)),
            pl.BlockSpec((_VT, _DIN), lambda v, b: (v, 0)),
            pl.BlockSpec((1, _VT), lambda v, b: (0, v)),
            pl.BlockSpec((1, _L, 1), lambda v, b: (b, 0, 0)),
        ],
        out_specs=pl.BlockSpec((1, _L, _VT), lambda v, b: (b, 0, v)),
        out_shape=jax.ShapeDtypeStruct((_B, _L, _USERS), f32),
    )(att_out, out_w, out_b.reshape(1, _USERS), seq_t)

    return out3.reshape(_B * _L, _USERS)


# speedup vs baseline: 5.5521x; 1.3448x over previous
import functools

import jax
import jax.numpy as jnp
from jax import lax
from jax.experimental import pallas as pl
from jax.experimental.pallas import tpu as pltpu
from jax.experimental.pallas import tpu_sc as plsc

_B = 16
_L = 49
_BL = _B * _L
_T = 8
_D = 64
_POS = 8
_DIN = _D + _POS
_H = 8
_DK = 64
_USERS = 100000
_STEP = 5
_NEG = -(2.0 ** 32) + 1.0

_ROWS = _T * _BL
_NW = 32
_CHUNK = 104
_PER_W = 208
_ROWS_PAD = _NW * _PER_W


def _sc_gather(table2d, idx):
    mesh = plsc.VectorSubcoreMesh(core_axis_name="c", subcore_axis_name="s")

    @functools.partial(
        pl.kernel,
        mesh=mesh,
        out_type=jax.ShapeDtypeStruct((_ROWS_PAD, _D), jnp.float32),
        compiler_params=pltpu.CompilerParams(use_tc_tiling_on_sc=False),
        scratch_types=[
            pltpu.VMEM((_CHUNK,), jnp.int32),
            pltpu.VMEM((_CHUNK, _D), jnp.float32),
            pltpu.SemaphoreType.DMA,
        ],
    )
    def gk(table_hbm, idx_hbm, out_hbm, idx_v, rows_v, sem):
        wid = lax.axis_index("s") * 2 + lax.axis_index("c")
        base = wid * _PER_W
        for c in range(_PER_W // _CHUNK):
            off = base + c * _CHUNK
            pltpu.sync_copy(idx_hbm.at[pl.ds(off, _CHUNK)], idx_v)
            pltpu.async_copy(table_hbm.at[idx_v], rows_v, sem).wait()
            pltpu.sync_copy(rows_v, out_hbm.at[pl.ds(off, _CHUNK)])

    return gk(table2d, idx)


def _attn_body(ts_ref, times_ref, dyu_ref, temb_ref, pos_ref, seqr_ref,
               wq_ref, wk_ref, wv_ref, wo_ref,
               l1w_ref, l1b_ref, l2w_ref, l2b_ref,
               s1_ref, b1_ref, s2_ref, b2_ref, out_ref):
    f32 = jnp.float32
    ts = ts_ref[...]
    times = times_ref[...]
    col = lax.broadcasted_iota(jnp.int32, (_B, _L), 1)
    valid = ts >= 0
    sentinel = jnp.int32(-2 ** 31)
    cur = jnp.max(times)
    oh_rows = []
    n_chunks = (_L + _STEP - 1) // _STEP
    for c in range(n_chunks):
        lo, hi = _STEP * c, min(_STEP * c + _STEP, _L)
        sel = (col >= lo) & (col < hi) & valid
        has_valid = jnp.any(sel)
        cmax = jnp.max(jnp.where(sel, ts, sentinel))
        cur = jnp.where(has_valid, cmax, cur)
        cnt = jnp.sum((times <= cur).astype(jnp.int32))
        val = jnp.where(cnt > 0, cnt - 1, jnp.int32(_T - 1))
        oh_rows.append(
            (lax.broadcasted_iota(jnp.int32, (1, _T), 1) == val).astype(f32))
    oh10 = jnp.concatenate(oh_rows, axis=0)
    r_iota = lax.broadcasted_iota(jnp.int32, (_BL, n_chunks), 0)
    c_iota = lax.broadcasted_iota(jnp.int32, (_BL, n_chunks), 1)
    exp_c = ((r_iota % _L) // _STEP == c_iota).astype(f32)
    t_sel = jnp.dot(oh10, temb_ref[...], preferred_element_type=f32)
    t_bl = jnp.dot(exp_c, t_sel, preferred_element_type=f32)

    dyu = dyu_ref[...]
    scale_t = 1.0 / (_D ** 0.5)
    affs = [jnp.sum(t_bl * dyu[t], axis=-1, keepdims=True) * scale_t
            for t in range(_T)]
    aff = jnp.concatenate(affs, axis=1)
    aff = aff - jnp.max(aff, axis=1, keepdims=True)
    ea = jnp.exp(aff)
    alpha = ea / jnp.sum(ea, axis=1, keepdims=True)
    dyemb = alpha[:, 0:1] * dyu[0]
    for t in range(1, _T):
        dyemb = dyemb + alpha[:, t:t + 1] * dyu[t]

    x = jnp.concatenate([dyemb, pos_ref[...]], axis=-1)

    seqr = seqr_ref[...]
    ri = lax.broadcasted_iota(jnp.int32, (_BL, _BL), 0)
    ci = lax.broadcasted_iota(jnp.int32, (_BL, _BL), 1)
    same_b = (ri // _L) == (ci // _L)
    inb_mask = ((ci % _L) > (ri % _L)) | (seqr == 0)
    neg_inf = f32(-jnp.inf)
    scale = 1.0 / (_DK ** 0.5 + 1e-6)
    vatt = jnp.zeros((_BL, _DIN), f32)
    for h in range(_H):
        qh = jnp.dot(x, wq_ref[h], preferred_element_type=f32)
        kh = jnp.dot(x, wk_ref[h], preferred_element_type=f32)
        vh = jnp.dot(x, wv_ref[h], preferred_element_type=f32)
        sc = lax.dot_general(qh, kh, (((1,), (1,)), ((), ())),
                             preferred_element_type=f32) * scale
        sc = jnp.where(same_b, jnp.where(inb_mask, f32(_NEG), sc), neg_inf)
        sc = sc - jnp.max(sc, axis=-1, keepdims=True)
        es = jnp.exp(sc)
        attn = es / jnp.sum(es, axis=-1, keepdims=True)
        ctx = jnp.dot(attn, vh, preferred_element_type=f32)
        vatt = vatt + jnp.dot(ctx, wo_ref[h], preferred_element_type=f32)

    def ln(v, s, b):
        mu = jnp.mean(v, axis=-1, keepdims=True)
        var = jnp.mean((v - mu) ** 2, axis=-1, keepdims=True)
        return (v - mu) / jnp.sqrt(var + 1e-5) * s + b

    x1 = ln(x + vatt, s1_ref[...], b1_ref[...])
    ffn = lax.dot_general(x1, l1w_ref[...], (((1,), (1,)), ((), ())),
                          preferred_element_type=f32) + l1b_ref[...]
    ffn = jnp.maximum(ffn, 0.0)
    ffn = lax.dot_general(ffn, l2w_ref[...], (((1,), (1,)), ((), ())),
                          preferred_element_type=f32) + l2b_ref[...]
    res = ln(x1 + ffn, s2_ref[...], b2_ref[...])
    out_ref[...] = res.astype(jnp.bfloat16).reshape(_B, _L, _DIN)


_VT = 8192
_NV = (_USERS + _VT - 1) // _VT


def _proj_body(att_ref, w_ref, b_ref, seq_ref, out_ref):
    f32 = jnp.float32
    bf16 = jnp.bfloat16
    vt = pl.program_id(0)
    att = att_ref[0]
    logits = lax.dot_general(att, w_ref[...], (((1,), (1,)), ((), ())),
                             preferred_element_type=f32) + b_ref[...]
    colv = lax.broadcasted_iota(jnp.int32, (_L, _VT), 1) + vt * _VT
    eq = (colv == seq_ref[0]).astype(bf16)
    qi = lax.broadcasted_iota(jnp.int32, (_L, _L), 0)
    ki = lax.broadcasted_iota(jnp.int32, (_L, _L), 1)
    tril = (ki <= qi).astype(bf16)
    pref = jnp.dot(tril, eq, preferred_element_type=f32)
    masked = (pref > 0.5) | (colv == 0)
    out_ref[...] = (logits + jnp.where(masked, f32(-1000.0), f32(0.0))
                    ).reshape(1, _L, _VT)


def kernel(input_seq, input_timestamp, tgt_idx, dyn_times, dyn_node_emb,
           time_emb, pos_emb, W_q, W_k, W_v, W_o, lin1_w, lin1_b, lin2_w,
           lin2_b, ln1_s, ln1_b, ln2_s, ln2_b, out_w, out_b):
    f32 = jnp.float32
    seq = input_seq[:, :-1].astype(jnp.int32)
    ts = input_timestamp[:, :-1].astype(jnp.int32)

    table2d = dyn_node_emb.reshape(_T * _USERS, _D)
    seq_flat = seq.reshape(-1)
    idx = (jnp.arange(_T, dtype=jnp.int32)[:, None] * _USERS
           + seq_flat[None, :]).reshape(-1)
    idx = jnp.concatenate(
        [idx, jnp.zeros((_ROWS_PAD - _ROWS,), jnp.int32)])
    rows = _sc_gather(table2d, idx)
    dyu = rows[:_ROWS].reshape(_T, _BL, _D)

    wq_h = W_q.reshape(_DIN, _H, _DK).transpose(1, 0, 2)
    wk_h = W_k.reshape(_DIN, _H, _DK).transpose(1, 0, 2)
    wv_h = W_v.reshape(_DIN, _H, _DK).transpose(1, 0, 2)
    wo_h = W_o.reshape(_H, _DK, _DIN)
    pos_bl = jnp.broadcast_to(pos_emb[None, :_L], (_B, _L, _POS)
                              ).reshape(_BL, _POS)
    att_out = pl.pallas_call(
        _attn_body,
        out_shape=jax.ShapeDtypeStruct((_B, _L, _DIN), jnp.bfloat16),
    )(ts, dyn_times.reshape(1, _T).astype(jnp.int32), dyu, time_emb,
      pos_bl, seq_flat.reshape(1, _BL),
      wq_h, wk_h, wv_h, wo_h,
      lin1_w, lin1_b.reshape(1, _DIN), lin2_w, lin2_b.reshape(1, _DIN),
      ln1_s.reshape(1, _DIN), ln1_b.reshape(1, _DIN),
      ln2_s.reshape(1, _DIN), ln2_b.reshape(1, _DIN))

    seq_t = seq.reshape(_B, _L, 1)
    out3 = pl.pallas_call(
        _proj_body,
        grid=(_NV, _B),
        in_specs=[
            pl.BlockSpec((1, _L, _DIN), lambda v, b: (b, 0, 0)),
            pl.BlockSpec((_VT, _DIN), lambda v, b: (v, 0)),
            pl.BlockSpec((1, _VT), lambda v, b: (0, v)),
            pl.BlockSpec((1, _L, 1), lambda v, b: (b, 0, 0)),
        ],
        out_specs=pl.BlockSpec((1, _L, _VT), lambda v, b: (b, 0, v)),
        out_shape=jax.ShapeDtypeStruct((_B, _L, _USERS), f32),
    )(att_out, out_w.astype(jnp.bfloat16), out_b.reshape(1, _USERS), seq_t)

    return out3.reshape(_BL, _USERS)


# speedup vs baseline: 7.3807x; 1.3294x over previous
import functools

import jax
import jax.numpy as jnp
from jax import lax
from jax.experimental import pallas as pl
from jax.experimental.pallas import tpu as pltpu
from jax.experimental.pallas import tpu_sc as plsc

_B = 16
_L = 49
_BL = _B * _L
_T = 8
_D = 64
_POS = 8
_DIN = _D + _POS
_H = 8
_DK = 64
_USERS = 100000
_STEP = 5
_NEG = -(2.0 ** 32) + 1.0

_ROWS = _T * _BL
_NW = 32
_CHUNK = 104
_PER_W = 208
_ROWS_PAD = _NW * _PER_W


def _sc_gather(table2d, idx):
    mesh = plsc.VectorSubcoreMesh(core_axis_name="c", subcore_axis_name="s")

    @functools.partial(
        pl.kernel,
        mesh=mesh,
        out_type=jax.ShapeDtypeStruct((_ROWS_PAD, _D), jnp.float32),
        compiler_params=pltpu.CompilerParams(use_tc_tiling_on_sc=False),
        scratch_types=[
            pltpu.VMEM((_CHUNK,), jnp.int32),
            pltpu.VMEM((_CHUNK, _D), jnp.float32),
            pltpu.SemaphoreType.DMA,
        ],
    )
    def gk(table_hbm, idx_hbm, out_hbm, idx_v, rows_v, sem):
        wid = lax.axis_index("s") * 2 + lax.axis_index("c")
        base = wid * _PER_W
        for c in range(_PER_W // _CHUNK):
            off = base + c * _CHUNK
            pltpu.sync_copy(idx_hbm.at[pl.ds(off, _CHUNK)], idx_v)
            pltpu.async_copy(table_hbm.at[idx_v], rows_v, sem).wait()
            pltpu.sync_copy(rows_v, out_hbm.at[pl.ds(off, _CHUNK)])

    return gk(table2d, idx)


def _attn_body(ts_ref, times_ref, dyu_ref, temb_ref, pos_ref, seqr_ref,
               wq_ref, wk_ref, wv_ref, wo_ref,
               l1w_ref, l1b_ref, l2w_ref, l2b_ref,
               s1_ref, b1_ref, s2_ref, b2_ref, out_ref):
    f32 = jnp.float32
    ts = ts_ref[...]
    times = times_ref[...]
    col = lax.broadcasted_iota(jnp.int32, (_B, _L), 1)
    valid = ts >= 0
    sentinel = jnp.int32(-2 ** 31)
    cur = jnp.max(times)
    oh_rows = []
    n_chunks = (_L + _STEP - 1) // _STEP
    for c in range(n_chunks):
        lo, hi = _STEP * c, min(_STEP * c + _STEP, _L)
        sel = (col >= lo) & (col < hi) & valid
        has_valid = jnp.any(sel)
        cmax = jnp.max(jnp.where(sel, ts, sentinel))
        cur = jnp.where(has_valid, cmax, cur)
        cnt = jnp.sum((times <= cur).astype(jnp.int32))
        val = jnp.where(cnt > 0, cnt - 1, jnp.int32(_T - 1))
        oh_rows.append(
            (lax.broadcasted_iota(jnp.int32, (1, _T), 1) == val).astype(f32))
    oh10 = jnp.concatenate(oh_rows, axis=0)
    r_iota = lax.broadcasted_iota(jnp.int32, (_BL, n_chunks), 0)
    c_iota = lax.broadcasted_iota(jnp.int32, (_BL, n_chunks), 1)
    exp_c = ((r_iota % _L) // _STEP == c_iota).astype(f32)
    t_sel = jnp.dot(oh10, temb_ref[...], preferred_element_type=f32)
    t_bl = jnp.dot(exp_c, t_sel, preferred_element_type=f32)

    dyu = dyu_ref[...]
    scale_t = 1.0 / (_D ** 0.5)
    affs = [jnp.sum(t_bl * dyu[t], axis=-1, keepdims=True) * scale_t
            for t in range(_T)]
    aff = jnp.concatenate(affs, axis=1)
    aff = aff - jnp.max(aff, axis=1, keepdims=True)
    ea = jnp.exp(aff)
    alpha = ea / jnp.sum(ea, axis=1, keepdims=True)
    dyemb = alpha[:, 0:1] * dyu[0]
    for t in range(1, _T):
        dyemb = dyemb + alpha[:, t:t + 1] * dyu[t]

    x = jnp.concatenate([dyemb, pos_ref[...]], axis=-1)

    seqr = seqr_ref[...]
    ri = lax.broadcasted_iota(jnp.int32, (_BL, _BL), 0)
    ci = lax.broadcasted_iota(jnp.int32, (_BL, _BL), 1)
    same_b = (ri // _L) == (ci // _L)
    inb_mask = ((ci % _L) > (ri % _L)) | (seqr == 0)
    neg_inf = f32(-jnp.inf)
    scale = 1.0 / (_DK ** 0.5 + 1e-6)
    vatt = jnp.zeros((_BL, _DIN), f32)
    for h in range(_H):
        qh = jnp.dot(x, wq_ref[h], preferred_element_type=f32)
        kh = jnp.dot(x, wk_ref[h], preferred_element_type=f32)
        vh = jnp.dot(x, wv_ref[h], preferred_element_type=f32)
        sc = lax.dot_general(qh, kh, (((1,), (1,)), ((), ())),
                             preferred_element_type=f32) * scale
        sc = jnp.where(same_b, jnp.where(inb_mask, f32(_NEG), sc), neg_inf)
        sc = sc - jnp.max(sc, axis=-1, keepdims=True)
        es = jnp.exp(sc)
        attn = es / jnp.sum(es, axis=-1, keepdims=True)
        ctx = jnp.dot(attn, vh, preferred_element_type=f32)
        vatt = vatt + jnp.dot(ctx, wo_ref[h], preferred_element_type=f32)

    def ln(v, s, b):
        mu = jnp.mean(v, axis=-1, keepdims=True)
        var = jnp.mean((v - mu) ** 2, axis=-1, keepdims=True)
        return (v - mu) / jnp.sqrt(var + 1e-5) * s + b

    x1 = ln(x + vatt, s1_ref[...], b1_ref[...])
    ffn = lax.dot_general(x1, l1w_ref[...], (((1,), (1,)), ((), ())),
                          preferred_element_type=f32) + l1b_ref[...]
    ffn = jnp.maximum(ffn, 0.0)
    ffn = lax.dot_general(ffn, l2w_ref[...], (((1,), (1,)), ((), ())),
                          preferred_element_type=f32) + l2b_ref[...]
    res = ln(x1 + ffn, s2_ref[...], b2_ref[...])
    out_ref[...] = res.astype(jnp.bfloat16)


_VT = 8192
_NV = (_USERS + _VT - 1) // _VT
_RB = 392
_NR = _BL // _RB
_BPR = _RB // _L


def _proj_body(att_ref, w_ref, b_ref, seq_ref, out_ref):
    f32 = jnp.float32
    bf16 = jnp.bfloat16
    vt = pl.program_id(0)
    qi = lax.broadcasted_iota(jnp.int32, (_L, _L), 0)
    ki = lax.broadcasted_iota(jnp.int32, (_L, _L), 1)
    tril = (ki <= qi).astype(bf16)
    colv = lax.broadcasted_iota(jnp.int32, (_L, _VT), 1) + vt * _VT
    for i in range(_BPR):
        att = att_ref[pl.ds(i * _L, _L)]
        logits = lax.dot_general(att, w_ref[...], (((1,), (1,)), ((), ())),
                                 preferred_element_type=f32) + b_ref[...]
        eq = (colv == seq_ref[pl.ds(i * _L, _L)]).astype(bf16)
        pref = jnp.dot(tril, eq, preferred_element_type=f32)
        masked = (pref > 0.5) | (colv == 0)
        out_ref[pl.ds(i * _L, _L)] = logits + jnp.where(
            masked, f32(-1000.0), f32(0.0))


def kernel(input_seq, input_timestamp, tgt_idx, dyn_times, dyn_node_emb,
           time_emb, pos_emb, W_q, W_k, W_v, W_o, lin1_w, lin1_b, lin2_w,
           lin2_b, ln1_s, ln1_b, ln2_s, ln2_b, out_w, out_b):
    f32 = jnp.float32
    seq = input_seq[:, :-1].astype(jnp.int32)
    ts = input_timestamp[:, :-1].astype(jnp.int32)

    table2d = dyn_node_emb.reshape(_T * _USERS, _D)
    seq_flat = seq.reshape(-1)
    idx = (jnp.arange(_T, dtype=jnp.int32)[:, None] * _USERS
           + seq_flat[None, :]).reshape(-1)
    idx = jnp.concatenate(
        [idx, jnp.zeros((_ROWS_PAD - _ROWS,), jnp.int32)])
    rows = _sc_gather(table2d, idx)
    dyu = rows[:_ROWS].reshape(_T, _BL, _D)

    wq_h = W_q.reshape(_DIN, _H, _DK).transpose(1, 0, 2)
    wk_h = W_k.reshape(_DIN, _H, _DK).transpose(1, 0, 2)
    wv_h = W_v.reshape(_DIN, _H, _DK).transpose(1, 0, 2)
    wo_h = W_o.reshape(_H, _DK, _DIN)
    pos_bl = jnp.broadcast_to(pos_emb[None, :_L], (_B, _L, _POS)
                              ).reshape(_BL, _POS)
    att_out = pl.pallas_call(
        _attn_body,
        out_shape=jax.ShapeDtypeStruct((_BL, _DIN), jnp.bfloat16),
    )(ts, dyn_times.reshape(1, _T).astype(jnp.int32), dyu, time_emb,
      pos_bl, seq_flat.reshape(1, _BL),
      wq_h, wk_h, wv_h, wo_h,
      lin1_w, lin1_b.reshape(1, _DIN), lin2_w, lin2_b.reshape(1, _DIN),
      ln1_s.reshape(1, _DIN), ln1_b.reshape(1, _DIN),
      ln2_s.reshape(1, _DIN), ln2_b.reshape(1, _DIN))

    seq_col = seq_flat.reshape(_BL, 1)
    out3 = pl.pallas_call(
        _proj_body,
        grid=(_NV, _NR),
        in_specs=[
            pl.BlockSpec((_RB, _DIN), lambda v, r: (r, 0)),
            pl.BlockSpec((_VT, _DIN), lambda v, r: (v, 0)),
            pl.BlockSpec((1, _VT), lambda v, r: (0, v)),
            pl.BlockSpec((_RB, 1), lambda v, r: (r, 0)),
        ],
        out_specs=pl.BlockSpec((_RB, _VT), lambda v, r: (r, v)),
        out_shape=jax.ShapeDtypeStruct((_BL, _USERS), f32),
    )(att_out, out_w.astype(jnp.bfloat16), out_b.reshape(1, _USERS), seq_col)

    return out3


# speedup vs baseline: 8.6810x; 1.1762x over previous
import functools

import jax
import jax.numpy as jnp
from jax import lax
from jax.experimental import pallas as pl
from jax.experimental.pallas import tpu as pltpu
from jax.experimental.pallas import tpu_sc as plsc

_B = 16
_L = 49
_BL = _B * _L
_T = 8
_D = 64
_POS = 8
_DIN = _D + _POS
_H = 8
_DK = 64
_USERS = 100000
_STEP = 5
_NEG = -(2.0 ** 32) + 1.0

_ROWS = _T * _BL
_NW = 32
_CHUNK = 104
_PER_W = 208
_ROWS_PAD = _NW * _PER_W


def _sc_gather(table2d, idx):
    mesh = plsc.VectorSubcoreMesh(core_axis_name="c", subcore_axis_name="s")

    @functools.partial(
        pl.kernel,
        mesh=mesh,
        out_type=jax.ShapeDtypeStruct((_ROWS_PAD, _D), jnp.float32),
        compiler_params=pltpu.CompilerParams(use_tc_tiling_on_sc=False),
        scratch_types=[
            pltpu.VMEM((_CHUNK,), jnp.int32),
            pltpu.VMEM((_CHUNK, _D), jnp.float32),
            pltpu.SemaphoreType.DMA,
        ],
    )
    def gk(table_hbm, idx_hbm, out_hbm, idx_v, rows_v, sem):
        wid = lax.axis_index("s") * 2 + lax.axis_index("c")
        base = wid * _PER_W
        for c in range(_PER_W // _CHUNK):
            off = base + c * _CHUNK
            pltpu.sync_copy(idx_hbm.at[pl.ds(off, _CHUNK)], idx_v)
            pltpu.async_copy(table_hbm.at[idx_v], rows_v, sem).wait()
            pltpu.sync_copy(rows_v, out_hbm.at[pl.ds(off, _CHUNK)])

    return gk(table2d, idx)


def _attn_body(ts_ref, times_ref, dyu_ref, temb_ref, pos_ref, seqr_ref,
               wq_ref, wk_ref, wv_ref, wo_ref,
               l1w_ref, l1b_ref, l2w_ref, l2b_ref,
               s1_ref, b1_ref, s2_ref, b2_ref, out_ref):
    f32 = jnp.float32
    ts = ts_ref[...]
    times = times_ref[...]
    col = lax.broadcasted_iota(jnp.int32, (_B, _L), 1)
    valid = ts >= 0
    sentinel = jnp.int32(-2 ** 31)
    cur = jnp.max(times)
    oh_rows = []
    n_chunks = (_L + _STEP - 1) // _STEP
    for c in range(n_chunks):
        lo, hi = _STEP * c, min(_STEP * c + _STEP, _L)
        sel = (col >= lo) & (col < hi) & valid
        has_valid = jnp.any(sel)
        cmax = jnp.max(jnp.where(sel, ts, sentinel))
        cur = jnp.where(has_valid, cmax, cur)
        cnt = jnp.sum((times <= cur).astype(jnp.int32))
        val = jnp.where(cnt > 0, cnt - 1, jnp.int32(_T - 1))
        oh_rows.append(
            (lax.broadcasted_iota(jnp.int32, (1, _T), 1) == val).astype(f32))
    oh10 = jnp.concatenate(oh_rows, axis=0)
    r_iota = lax.broadcasted_iota(jnp.int32, (_BL, n_chunks), 0)
    c_iota = lax.broadcasted_iota(jnp.int32, (_BL, n_chunks), 1)
    exp_c = ((r_iota % _L) // _STEP == c_iota).astype(f32)
    t_sel = jnp.dot(oh10, temb_ref[...], preferred_element_type=f32)
    t_bl = jnp.dot(exp_c, t_sel, preferred_element_type=f32)

    dyu = dyu_ref[...]
    scale_t = 1.0 / (_D ** 0.5)
    affs = [jnp.sum(t_bl * dyu[t], axis=-1, keepdims=True) * scale_t
            for t in range(_T)]
    aff = jnp.concatenate(affs, axis=1)
    aff = aff - jnp.max(aff, axis=1, keepdims=True)
    ea = jnp.exp(aff)
    alpha = ea / jnp.sum(ea, axis=1, keepdims=True)
    dyemb = alpha[:, 0:1] * dyu[0]
    for t in range(1, _T):
        dyemb = dyemb + alpha[:, t:t + 1] * dyu[t]

    x = jnp.concatenate([dyemb, pos_ref[...]], axis=-1)

    seqr = seqr_ref[...]
    ri = lax.broadcasted_iota(jnp.int32, (_BL, _BL), 0)
    ci = lax.broadcasted_iota(jnp.int32, (_BL, _BL), 1)
    same_b = (ri // _L) == (ci // _L)
    inb_mask = ((ci % _L) > (ri % _L)) | (seqr == 0)
    neg_inf = f32(-jnp.inf)
    scale = 1.0 / (_DK ** 0.5 + 1e-6)
    vatt = jnp.zeros((_BL, _DIN), f32)
    for h in range(_H):
        qh = jnp.dot(x, wq_ref[h], preferred_element_type=f32)
        kh = jnp.dot(x, wk_ref[h], preferred_element_type=f32)
        vh = jnp.dot(x, wv_ref[h], preferred_element_type=f32)
        sc = lax.dot_general(qh, kh, (((1,), (1,)), ((), ())),
                             preferred_element_type=f32) * scale
        sc = jnp.where(same_b, jnp.where(inb_mask, f32(_NEG), sc), neg_inf)
        sc = sc - jnp.max(sc, axis=-1, keepdims=True)
        es = jnp.exp(sc)
        attn = es / jnp.sum(es, axis=-1, keepdims=True)
        ctx = jnp.dot(attn, vh, preferred_element_type=f32)
        vatt = vatt + jnp.dot(ctx, wo_ref[h], preferred_element_type=f32)

    def ln(v, s, b):
        mu = jnp.mean(v, axis=-1, keepdims=True)
        var = jnp.mean((v - mu) ** 2, axis=-1, keepdims=True)
        return (v - mu) / jnp.sqrt(var + 1e-5) * s + b

    x1 = ln(x + vatt, s1_ref[...], b1_ref[...])
    ffn = lax.dot_general(x1, l1w_ref[...], (((1,), (1,)), ((), ())),
                          preferred_element_type=f32) + l1b_ref[...]
    ffn = jnp.maximum(ffn, 0.0)
    ffn = lax.dot_general(ffn, l2w_ref[...], (((1,), (1,)), ((), ())),
                          preferred_element_type=f32) + l2b_ref[...]
    res = ln(x1 + ffn, s2_ref[...], b2_ref[...])
    out_ref[...] = res.astype(jnp.bfloat16)


_VT = 8192
_NV = (_USERS + _VT - 1) // _VT
_RB = 392
_NR = _BL // _RB
_BPR = _RB // _L


def _proj_body(att_ref, w_ref, b_ref, seq_ref, out_ref):
    f32 = jnp.float32
    bf16 = jnp.bfloat16
    vt = pl.program_id(0)
    qi = lax.broadcasted_iota(jnp.int32, (_L, _L), 0)
    ki = lax.broadcasted_iota(jnp.int32, (_L, _L), 1)
    tril = (ki <= qi).astype(bf16)
    colv = lax.broadcasted_iota(jnp.int32, (_L, _VT), 1) + vt * _VT
    for i in range(_BPR):
        att = att_ref[pl.ds(i * _L, _L)]
        logits = jnp.dot(att, w_ref[...].astype(bf16),
                         preferred_element_type=f32) + b_ref[...]
        eq = (colv == seq_ref[pl.ds(i * _L, _L)]).astype(bf16)
        pref = jnp.dot(tril, eq, preferred_element_type=f32)
        masked = (pref > 0.5) | (colv == 0)
        out_ref[pl.ds(i * _L, _L)] = logits + jnp.where(
            masked, f32(-1000.0), f32(0.0))


def kernel(input_seq, input_timestamp, tgt_idx, dyn_times, dyn_node_emb,
           time_emb, pos_emb, W_q, W_k, W_v, W_o, lin1_w, lin1_b, lin2_w,
           lin2_b, ln1_s, ln1_b, ln2_s, ln2_b, out_w, out_b):
    f32 = jnp.float32
    seq = input_seq[:, :-1].astype(jnp.int32)
    ts = input_timestamp[:, :-1].astype(jnp.int32)

    table2d = dyn_node_emb.reshape(_T * _USERS, _D)
    seq_flat = seq.reshape(-1)
    idx = (jnp.arange(_T, dtype=jnp.int32)[:, None] * _USERS
           + seq_flat[None, :]).reshape(-1)
    idx = jnp.concatenate(
        [idx, jnp.zeros((_ROWS_PAD - _ROWS,), jnp.int32)])
    rows = _sc_gather(table2d, idx)
    dyu = rows[:_ROWS].reshape(_T, _BL, _D)

    wq_h = W_q.reshape(_DIN, _H, _DK).transpose(1, 0, 2)
    wk_h = W_k.reshape(_DIN, _H, _DK).transpose(1, 0, 2)
    wv_h = W_v.reshape(_DIN, _H, _DK).transpose(1, 0, 2)
    wo_h = W_o.reshape(_H, _DK, _DIN)
    pos_bl = jnp.broadcast_to(pos_emb[None, :_L], (_B, _L, _POS)
                              ).reshape(_BL, _POS)
    att_out = pl.pallas_call(
        _attn_body,
        out_shape=jax.ShapeDtypeStruct((_BL, _DIN), jnp.bfloat16),
    )(ts, dyn_times.reshape(1, _T).astype(jnp.int32), dyu, time_emb,
      pos_bl, seq_flat.reshape(1, _BL),
      wq_h, wk_h, wv_h, wo_h,
      lin1_w, lin1_b.reshape(1, _DIN), lin2_w, lin2_b.reshape(1, _DIN),
      ln1_s.reshape(1, _DIN), ln1_b.reshape(1, _DIN),
      ln2_s.reshape(1, _DIN), ln2_b.reshape(1, _DIN))

    seq_col = seq_flat.reshape(_BL, 1)
    out3 = pl.pallas_call(
        _proj_body,
        grid=(_NV, _NR),
        compiler_params=pltpu.CompilerParams(
            dimension_semantics=("parallel", "parallel")),
        in_specs=[
            pl.BlockSpec((_RB, _DIN), lambda v, r: (r, 0)),
            pl.BlockSpec((_DIN, _VT), lambda v, r: (0, v)),
            pl.BlockSpec((1, _VT), lambda v, r: (0, v)),
            pl.BlockSpec((_RB, 1), lambda v, r: (r, 0)),
        ],
        out_specs=pl.BlockSpec((_RB, _VT), lambda v, r: (r, v)),
        out_shape=jax.ShapeDtypeStruct((_BL, _USERS), f32),
    )(att_out, out_w.T, out_b.reshape(1, _USERS), seq_col)

    return out3
